# baseline (speedup 1.0000x reference)
import functools

import jax
import jax.numpy as jnp
from jax.experimental import pallas as pl
from jax.experimental.pallas import tpu as pltpu
from jax.experimental.pallas import tpu_sc as plsc

T = 2048
D = 4096
F = 14336
E = 8
RB = 1024
SUB = 128
NB = (2 * T) // RB + (E - 1)
FB = 256
FK = 256
NEG = -1e30


TBR = 256


def _router1_kernel(x_ref, gw_ref, a1_ref, a2_ref, g_ref):
    logits = jax.lax.dot_general(
        x_ref[...], gw_ref[...], (((1,), (0,)), ((), ())),
        precision=jax.lax.Precision.DEFAULT,
        preferred_element_type=jnp.float32)
    iota_e = jax.lax.broadcasted_iota(jnp.int32, (TBR, E), 1)
    l1 = jnp.max(logits, axis=1, keepdims=True)
    i1 = jnp.min(jnp.where(logits == l1, iota_e, E), axis=1,
                 keepdims=True)
    masked = jnp.where(iota_e == i1, NEG, logits)
    l2 = jnp.max(masked, axis=1, keepdims=True)
    i2 = jnp.min(jnp.where(masked == l2, iota_e, E), axis=1, keepdims=True)
    g_ref[:, 0:1] = jax.nn.sigmoid(l1 - l2)
    g_ref[:, 1:2] = jax.nn.sigmoid(l2 - l1)
    a1_ref[...] = (iota_e == i1).astype(jnp.int32)
    a2_ref[...] = (iota_e == i2).astype(jnp.int32)


def _run_router1(x, gate_w):
    out_shapes = (
        jax.ShapeDtypeStruct((T, E), jnp.int32),
        jax.ShapeDtypeStruct((T, E), jnp.int32),
        jax.ShapeDtypeStruct((T, 2), jnp.float32),
    )
    return pl.pallas_call(
        _router1_kernel,
        grid=(T // TBR,),
        out_shape=out_shapes,
        in_specs=[
            pl.BlockSpec((TBR, D), lambda i: (i, 0)),
            pl.BlockSpec((D, E), lambda i: (0, 0)),
        ],
        out_specs=(
            pl.BlockSpec((TBR, E), lambda i: (i, 0)),
            pl.BlockSpec((TBR, E), lambda i: (i, 0)),
            pl.BlockSpec((TBR, 2), lambda i: (i, 0)),
        ),
        compiler_params=pltpu.CompilerParams(
            dimension_semantics=("arbitrary",)),
    )(x, gate_w)


def _router2_kernel(a1_ref, a2_ref, be_ref, bv_ref, dest_ref,
                    cnt_vmem, cnt_smem, sem):
    a1 = a1_ref[...]
    a2 = a2_ref[...]
    a = a1 + a2
    s = a
    sh = 1
    while sh < T:
        s = s + jnp.concatenate(
            [jnp.zeros((sh, E), jnp.int32), s[:T - sh, :]], axis=0)
        sh *= 2
    pos = s - a
    counts = s[T - 1:T, :]
    cnt_vmem[...] = counts
    copy = pltpu.make_async_copy(cnt_vmem, cnt_smem, sem)
    copy.start()
    copy.wait()

    def expert_step(e, blk):
        cnt = cnt_smem[0, e]
        nb = (cnt + RB - 1) // RB

        def blk_step(j, _):
            be_ref[blk + j] = e
            bv_ref[blk + j] = jnp.minimum(RB, cnt - j * RB)
            return 0
        jax.lax.fori_loop(0, nb, blk_step, 0, unroll=False)
        return blk + nb

    used = jax.lax.fori_loop(0, E, expert_step, 0, unroll=False)

    def pad_step(j, _):
        @pl.when(j >= used)
        def _():
            be_ref[j] = E - 1
            bv_ref[j] = 0
        return 0
    jax.lax.fori_loop(0, NB, pad_step, 0, unroll=False)

    nb_vec = (counts + (RB - 1)) // RB
    off = nb_vec
    sh = 1
    while sh < E:
        off = off + jnp.concatenate(
            [jnp.zeros((1, sh), jnp.int32), off[:, :E - sh]], axis=1)
        sh *= 2
    pado = (off - nb_vec) * RB

    dest_ref[:, 0:1] = jnp.sum(a1 * (pos + pado), axis=1, keepdims=True)
    dest_ref[:, 1:2] = jnp.sum(a2 * (pos + pado), axis=1, keepdims=True)


def _run_router2(a1, a2):
    out_shapes = (
        jax.ShapeDtypeStruct((NB,), jnp.int32),
        jax.ShapeDtypeStruct((NB,), jnp.int32),
        jax.ShapeDtypeStruct((T, 2), jnp.int32),
    )
    return pl.pallas_call(
        _router2_kernel,
        out_shape=out_shapes,
        in_specs=[
            pl.BlockSpec(memory_space=pltpu.VMEM),
            pl.BlockSpec(memory_space=pltpu.VMEM),
        ],
        out_specs=(
            pl.BlockSpec(memory_space=pltpu.SMEM),
            pl.BlockSpec(memory_space=pltpu.SMEM),
            pl.BlockSpec(memory_space=pltpu.VMEM),
        ),
        scratch_shapes=[
            pltpu.VMEM((1, E), jnp.int32),
            pltpu.SMEM((1, E), jnp.int32),
            pltpu.SemaphoreType.DMA,
        ],
    )(a1, a2)


def _run_router(x, gate_w):
    a1, a2, gws = _run_router1(x, gate_w)
    be, bv, dest = _run_router2(a1, a2)
    return be, bv, dest, gws


def _sc_mesh():
    return plsc.VectorSubcoreMesh(core_axis_name="core",
                                  subcore_axis_name="subcore",
                                  num_cores=2, num_subcores=16)


WIN = 128
DCS = 256
QCS = D // DCS
DCC = 256
QCC = D // DCC


def _run_dispatch(x_words, idxs):
    @functools.partial(
        pl.kernel,
        out_type=jax.ShapeDtypeStruct((NB * RB * QCS, DCS), jnp.float32),
        mesh=_sc_mesh())
    def scatter_kernel(x_hbm, i_hbm, o_hbm):
        def body(x_vmem, i_vmem):
            pltpu.sync_copy(x_vmem, o_hbm.at[i_vmem.at[0]])

        pltpu.emit_pipeline(
            body,
            grid=(QCS, (2 * T) // WIN),
            in_specs=[
                pl.BlockSpec((WIN, DCS),
                             index_map=lambda q, i: (i % (T // WIN), q)),
                pl.BlockSpec((1, WIN), index_map=lambda q, i: (q, i)),
            ],
            out_specs=[],
            core_axis_name=("core", "subcore"),
            dimension_semantics=(pltpu.PARALLEL, pltpu.PARALLEL),
        )(x_hbm, i_hbm)

    return scatter_kernel(x_words, idxs)


def _run_combine(y_view, i1q, i2q, g1, g2):
    @functools.partial(
        pl.kernel,
        out_type=jax.ShapeDtypeStruct((T, D), jnp.float32),
        mesh=_sc_mesh(),
        scratch_types=[pltpu.VMEM((WIN, DCC), jnp.float32)])
    def combine_kernel(y_hbm, d1_hbm, d2_hbm, g1_hbm, g2_hbm, o_hbm, tmp):
        def body(i1_vmem, i2_vmem, g1_vmem, g2_vmem, o_vmem):
            pltpu.sync_copy(y_hbm.at[i1_vmem.at[0]], o_vmem)
            pltpu.sync_copy(y_hbm.at[i2_vmem.at[0]], tmp)

            @pl.loop(0, WIN)
            def _(i):
                ga = g1_vmem[0, pl.ds(i, 1)][0]
                gb = g2_vmem[0, pl.ds(i, 1)][0]

                @pl.loop(0, DCC, step=16)
                def _(c):
                    o_vmem.at[i, pl.ds(c, 16)][...] = (
                        ga * o_vmem.at[i, pl.ds(c, 16)][...]
                        + gb * tmp.at[i, pl.ds(c, 16)][...])

        iq_spec = pl.BlockSpec((1, WIN), index_map=lambda q, i: (q, i))
        g_spec = pl.BlockSpec((1, WIN), index_map=lambda q, i: (0, i))
        pltpu.emit_pipeline(
            body,
            grid=(QCC, T // WIN),
            in_specs=[iq_spec, iq_spec, g_spec, g_spec],
            out_specs=[pl.BlockSpec((WIN, DCC),
                                    index_map=lambda q, i: (i, q))],
            core_axis_name=("core", "subcore"),
            dimension_semantics=(pltpu.PARALLEL, pltpu.PARALLEL),
        )(d1_hbm, d2_hbm, g1_hbm, g2_hbm, o_hbm)

    return combine_kernel(y_view, i1q, i2q, g1, g2)


def _cast_x_kernel(x_ref, o_ref):
    o_ref[...] = x_ref[...].astype(jnp.bfloat16)


def _run_cast_x(x_sorted):
    return pl.pallas_call(
        _cast_x_kernel,
        grid=(NB * 2,),
        in_specs=[pl.BlockSpec((RB // 2, D), lambda i: (i, 0))],
        out_specs=pl.BlockSpec((RB // 2, D), lambda i: (i, 0)),
        out_shape=jax.ShapeDtypeStruct((NB * RB, D), jnp.bfloat16),
        compiler_params=pltpu.CompilerParams(
            dimension_semantics=("arbitrary",)),
    )(x_sorted)


NF1 = F // FB
NF2 = F // FK


def _ffn_in_kernel(be_ref, bv_ref, x_ref, w1_ref, w3_ref, h_ref,
                   w1b_ref, w3b_ref):
    b = pl.program_id(0)
    f = pl.program_id(1)
    valid = bv_ref[b]
    cur = jax.lax.rem(f, 2)
    prv = jax.lax.rem(f + 1, 2)

    @pl.when(f < NF1)
    def _():
        w1b_ref[cur] = w1_ref[0].astype(jnp.bfloat16)
        w3b_ref[cur] = w3_ref[0].astype(jnp.bfloat16)

    for s in range(RB // SUB):
        @pl.when((f > 0) & (s * SUB < valid))
        def _():
            xs = x_ref[s * SUB:(s + 1) * SUB, :]
            a = jax.lax.dot_general(
                xs, w1b_ref[prv], (((1,), (0,)), ((), ())),
                preferred_element_type=jnp.float32)
            u = jax.lax.dot_general(
                xs, w3b_ref[prv], (((1,), (0,)), ((), ())),
                preferred_element_type=jnp.float32)
            h = (a * jax.nn.sigmoid(a)) * u
            h_ref[s * SUB:(s + 1) * SUB, :] = h.astype(jnp.bfloat16)


def _run_ffn_in(x_bf, w1, w3, be, bv):
    grid = (NB, NF1 + 1)

    def x_map(b, f, be_r, bv_r):
        return (jnp.where(bv_r[b] > 0, b, 0), 0)

    def w_map(b, f, be_r, bv_r):
        fc = jnp.minimum(f, NF1 - 1)
        return (be_r[b], 0, jnp.where(bv_r[b] > 0, fc, 0))

    def h_map(b, f, be_r, bv_r):
        return (b, jnp.maximum(f - 1, 0))

    return pl.pallas_call(
        _ffn_in_kernel,
        grid_spec=pltpu.PrefetchScalarGridSpec(
            num_scalar_prefetch=2,
            grid=grid,
            in_specs=[
                pl.BlockSpec((RB, D), x_map),
                pl.BlockSpec((1, D, FB), w_map),
                pl.BlockSpec((1, D, FB), w_map),
            ],
            out_specs=pl.BlockSpec((RB, FB), h_map),
            scratch_shapes=[
                pltpu.VMEM((2, D, FB), jnp.bfloat16),
                pltpu.VMEM((2, D, FB), jnp.bfloat16),
            ],
        ),
        out_shape=jax.ShapeDtypeStruct((NB * RB, F), jnp.bfloat16),
        compiler_params=pltpu.CompilerParams(
            dimension_semantics=("arbitrary", "arbitrary")),
    )(be, bv, x_bf, w1, w3)


def _ffn_out_kernel(be_ref, bv_ref, h_ref, w2_ref, y_ref, w2b_ref):
    b = pl.program_id(0)
    k = pl.program_id(1)
    valid = bv_ref[b]
    cur = jax.lax.rem(k, 2)
    prv = jax.lax.rem(k + 1, 2)

    @pl.when(k < NF2)
    def _():
        w2b_ref[cur] = w2_ref[0].astype(jnp.bfloat16)

    for s in range(RB // SUB):
        @pl.when((k > 0) & (s * SUB < valid))
        def _():
            hs = h_ref[s * SUB:(s + 1) * SUB, :]
            part = jax.lax.dot_general(
                hs, w2b_ref[prv], (((1,), (0,)), ((), ())),
                preferred_element_type=jnp.float32)
            prev = y_ref[s * SUB:(s + 1) * SUB, :]
            y_ref[s * SUB:(s + 1) * SUB, :] = jnp.where(k == 1, part,
                                                        prev + part)


def _run_ffn_out(h, w2, be, bv):
    grid = (NB, NF2 + 1)

    def h_map(b, k, be_r, bv_r):
        kc = jnp.maximum(k - 1, 0)
        return (b, jnp.where(bv_r[b] > 0, kc, 0))

    def w_map(b, k, be_r, bv_r):
        kc = jnp.minimum(k, NF2 - 1)
        return (be_r[b], jnp.where(bv_r[b] > 0, kc, 0), 0)

    def y_map(b, k, be_r, bv_r):
        return (b, 0)

    return pl.pallas_call(
        _ffn_out_kernel,
        grid_spec=pltpu.PrefetchScalarGridSpec(
            num_scalar_prefetch=2,
            grid=grid,
            in_specs=[
                pl.BlockSpec((RB, FK), h_map),
                pl.BlockSpec((1, FK, D), w_map),
            ],
            out_specs=pl.BlockSpec((RB, D), y_map),
            scratch_shapes=[
                pltpu.VMEM((2, FK, D), jnp.bfloat16),
            ],
        ),
        out_shape=jax.ShapeDtypeStruct((NB * RB, D), jnp.float32),
        compiler_params=pltpu.CompilerParams(
            dimension_semantics=("arbitrary", "arbitrary")),
    )(be, bv, h, w2)


def kernel(x, gate_w, w1, w3, w2):
    be, bv, dest, gws = _run_router(x, gate_w)
    d1 = dest[:, 0].reshape(1, T)
    d2 = dest[:, 1].reshape(1, T)
    g1 = gws[:, 0].reshape(1, T)
    g2 = gws[:, 1].reshape(1, T)
    dest_all = jnp.concatenate([d1, d2], axis=1)
    disp_idx = dest_all * QCS + jnp.arange(QCS, dtype=jnp.int32)[:, None]
    i1q = d1 * QCC + jnp.arange(QCC, dtype=jnp.int32)[:, None]
    i2q = d2 * QCC + jnp.arange(QCC, dtype=jnp.int32)[:, None]

    x_sorted = _run_dispatch(x, disp_idx)
    x_bf = x_sorted.reshape(NB * RB, D).astype(jnp.bfloat16)
    h = _run_ffn_in(x_bf, w1, w3, be, bv)
    y_sorted = _run_ffn_out(h, w2, be, bv)
    out = _run_combine(y_sorted.reshape(NB * RB * QCC, DCC), i1q, i2q, g1, g2)
    return out


# speedup vs baseline: 1.0983x; 1.0983x over previous
import functools

import jax
import jax.numpy as jnp
from jax.experimental import pallas as pl
from jax.experimental.pallas import tpu as pltpu
from jax.experimental.pallas import tpu_sc as plsc

T = 2048
D = 4096
F = 14336
E = 8
RB = 1024
SUB = 256
NB = (2 * T) // RB + (E - 1)
FB = 256
FK = 256
NEG = -1e30


TBR = 256


def _router1_kernel(x_ref, gw_ref, a1_ref, a2_ref, g_ref):
    logits = jax.lax.dot_general(
        x_ref[...], gw_ref[...], (((1,), (0,)), ((), ())),
        precision=jax.lax.Precision.DEFAULT,
        preferred_element_type=jnp.float32)
    iota_e = jax.lax.broadcasted_iota(jnp.int32, (TBR, E), 1)
    l1 = jnp.max(logits, axis=1, keepdims=True)
    i1 = jnp.min(jnp.where(logits == l1, iota_e, E), axis=1,
                 keepdims=True)
    masked = jnp.where(iota_e == i1, NEG, logits)
    l2 = jnp.max(masked, axis=1, keepdims=True)
    i2 = jnp.min(jnp.where(masked == l2, iota_e, E), axis=1, keepdims=True)
    g_ref[:, 0:1] = jax.nn.sigmoid(l1 - l2)
    g_ref[:, 1:2] = jax.nn.sigmoid(l2 - l1)
    a1_ref[...] = (iota_e == i1).astype(jnp.int32)
    a2_ref[...] = (iota_e == i2).astype(jnp.int32)


def _run_router1(x, gate_w):
    out_shapes = (
        jax.ShapeDtypeStruct((T, E), jnp.int32),
        jax.ShapeDtypeStruct((T, E), jnp.int32),
        jax.ShapeDtypeStruct((T, 2), jnp.float32),
    )
    return pl.pallas_call(
        _router1_kernel,
        grid=(T // TBR,),
        out_shape=out_shapes,
        in_specs=[
            pl.BlockSpec((TBR, D), lambda i: (i, 0)),
            pl.BlockSpec((D, E), lambda i: (0, 0)),
        ],
        out_specs=(
            pl.BlockSpec((TBR, E), lambda i: (i, 0)),
            pl.BlockSpec((TBR, E), lambda i: (i, 0)),
            pl.BlockSpec((TBR, 2), lambda i: (i, 0)),
        ),
        compiler_params=pltpu.CompilerParams(
            dimension_semantics=("arbitrary",)),
    )(x, gate_w)


def _router2_kernel(a1_ref, a2_ref, be_ref, bv_ref, dest_ref,
                    cnt_vmem, cnt_smem, sem):
    a1 = a1_ref[...]
    a2 = a2_ref[...]
    a = a1 + a2
    s = a
    sh = 1
    while sh < T:
        s = s + jnp.concatenate(
            [jnp.zeros((sh, E), jnp.int32), s[:T - sh, :]], axis=0)
        sh *= 2
    pos = s - a
    counts = s[T - 1:T, :]
    cnt_vmem[...] = counts
    copy = pltpu.make_async_copy(cnt_vmem, cnt_smem, sem)
    copy.start()
    copy.wait()

    def expert_step(e, blk):
        cnt = cnt_smem[0, e]
        nb = (cnt + RB - 1) // RB

        def blk_step(j, _):
            be_ref[blk + j] = e
            bv_ref[blk + j] = jnp.minimum(RB, cnt - j * RB)
            return 0
        jax.lax.fori_loop(0, nb, blk_step, 0, unroll=False)
        return blk + nb

    used = jax.lax.fori_loop(0, E, expert_step, 0, unroll=False)

    def pad_step(j, _):
        @pl.when(j >= used)
        def _():
            be_ref[j] = E - 1
            bv_ref[j] = 0
        return 0
    jax.lax.fori_loop(0, NB, pad_step, 0, unroll=False)

    nb_vec = (counts + (RB - 1)) // RB
    off = nb_vec
    sh = 1
    while sh < E:
        off = off + jnp.concatenate(
            [jnp.zeros((1, sh), jnp.int32), off[:, :E - sh]], axis=1)
        sh *= 2
    pado = (off - nb_vec) * RB

    dest_ref[:, 0:1] = jnp.sum(a1 * (pos + pado), axis=1, keepdims=True)
    dest_ref[:, 1:2] = jnp.sum(a2 * (pos + pado), axis=1, keepdims=True)


def _run_router2(a1, a2):
    out_shapes = (
        jax.ShapeDtypeStruct((NB,), jnp.int32),
        jax.ShapeDtypeStruct((NB,), jnp.int32),
        jax.ShapeDtypeStruct((T, 2), jnp.int32),
    )
    return pl.pallas_call(
        _router2_kernel,
        out_shape=out_shapes,
        in_specs=[
            pl.BlockSpec(memory_space=pltpu.VMEM),
            pl.BlockSpec(memory_space=pltpu.VMEM),
        ],
        out_specs=(
            pl.BlockSpec(memory_space=pltpu.SMEM),
            pl.BlockSpec(memory_space=pltpu.SMEM),
            pl.BlockSpec(memory_space=pltpu.VMEM),
        ),
        scratch_shapes=[
            pltpu.VMEM((1, E), jnp.int32),
            pltpu.SMEM((1, E), jnp.int32),
            pltpu.SemaphoreType.DMA,
        ],
    )(a1, a2)


def _run_router(x, gate_w):
    a1, a2, gws = _run_router1(x, gate_w)
    be, bv, dest = _run_router2(a1, a2)
    return be, bv, dest, gws


def _sc_mesh():
    return plsc.VectorSubcoreMesh(core_axis_name="core",
                                  subcore_axis_name="subcore",
                                  num_cores=2, num_subcores=16)


WIN = 128
DCS = 256
QCS = D // DCS
DCC = 256
QCC = D // DCC


def _run_dispatch(x_words, idxs):
    @functools.partial(
        pl.kernel,
        out_type=jax.ShapeDtypeStruct((NB * RB * QCS, DCS), jnp.float32),
        mesh=_sc_mesh())
    def scatter_kernel(x_hbm, i_hbm, o_hbm):
        def body(x_vmem, i_vmem):
            pltpu.sync_copy(x_vmem, o_hbm.at[i_vmem.at[0]])

        pltpu.emit_pipeline(
            body,
            grid=(QCS, (2 * T) // WIN),
            in_specs=[
                pl.BlockSpec((WIN, DCS),
                             index_map=lambda q, i: (i % (T // WIN), q)),
                pl.BlockSpec((1, WIN), index_map=lambda q, i: (q, i)),
            ],
            out_specs=[],
            core_axis_name=("core", "subcore"),
            dimension_semantics=(pltpu.PARALLEL, pltpu.PARALLEL),
        )(x_hbm, i_hbm)

    return scatter_kernel(x_words, idxs)


def _run_combine(y_view, i1q, i2q, g1, g2):
    @functools.partial(
        pl.kernel,
        out_type=jax.ShapeDtypeStruct((T, D), jnp.float32),
        mesh=_sc_mesh(),
        scratch_types=[pltpu.VMEM((WIN, DCC), jnp.float32)])
    def combine_kernel(y_hbm, d1_hbm, d2_hbm, g1_hbm, g2_hbm, o_hbm, tmp):
        def body(i1_vmem, i2_vmem, g1_vmem, g2_vmem, o_vmem):
            pltpu.sync_copy(y_hbm.at[i1_vmem.at[0]], o_vmem)
            pltpu.sync_copy(y_hbm.at[i2_vmem.at[0]], tmp)

            @pl.loop(0, WIN)
            def _(i):
                ga = g1_vmem[0, pl.ds(i, 1)][0]
                gb = g2_vmem[0, pl.ds(i, 1)][0]

                @pl.loop(0, DCC, step=16)
                def _(c):
                    o_vmem.at[i, pl.ds(c, 16)][...] = (
                        ga * o_vmem.at[i, pl.ds(c, 16)][...]
                        + gb * tmp.at[i, pl.ds(c, 16)][...])

        iq_spec = pl.BlockSpec((1, WIN), index_map=lambda q, i: (q, i))
        g_spec = pl.BlockSpec((1, WIN), index_map=lambda q, i: (0, i))
        pltpu.emit_pipeline(
            body,
            grid=(QCC, T // WIN),
            in_specs=[iq_spec, iq_spec, g_spec, g_spec],
            out_specs=[pl.BlockSpec((WIN, DCC),
                                    index_map=lambda q, i: (i, q))],
            core_axis_name=("core", "subcore"),
            dimension_semantics=(pltpu.PARALLEL, pltpu.PARALLEL),
        )(d1_hbm, d2_hbm, g1_hbm, g2_hbm, o_hbm)

    return combine_kernel(y_view, i1q, i2q, g1, g2)


def _cast_x_kernel(x_ref, o_ref):
    o_ref[...] = x_ref[...].astype(jnp.bfloat16)


def _run_cast_x(x_sorted):
    return pl.pallas_call(
        _cast_x_kernel,
        grid=(NB * 2,),
        in_specs=[pl.BlockSpec((RB // 2, D), lambda i: (i, 0))],
        out_specs=pl.BlockSpec((RB // 2, D), lambda i: (i, 0)),
        out_shape=jax.ShapeDtypeStruct((NB * RB, D), jnp.bfloat16),
        compiler_params=pltpu.CompilerParams(
            dimension_semantics=("arbitrary",)),
    )(x_sorted)


NF1 = F // FB
NF2 = F // FK


def _ffn_in_kernel(be_ref, bv_ref, x_ref, w1_ref, w3_ref, h_ref,
                   w1b_ref, w3b_ref):
    b = pl.program_id(0)
    f = pl.program_id(1)
    valid = bv_ref[b]
    cur = jax.lax.rem(f, 2)
    prv = jax.lax.rem(f + 1, 2)

    @pl.when(f < NF1)
    def _():
        w1b_ref[cur] = w1_ref[0].astype(jnp.bfloat16)
        w3b_ref[cur] = w3_ref[0].astype(jnp.bfloat16)

    for s in range(RB // SUB):
        @pl.when((f > 0) & (s * SUB < valid))
        def _():
            xs = x_ref[s * SUB:(s + 1) * SUB, :]
            a = jax.lax.dot_general(
                xs, w1b_ref[prv], (((1,), (0,)), ((), ())),
                preferred_element_type=jnp.float32)
            u = jax.lax.dot_general(
                xs, w3b_ref[prv], (((1,), (0,)), ((), ())),
                preferred_element_type=jnp.float32)
            h = (a * jax.nn.sigmoid(a)) * u
            h_ref[s * SUB:(s + 1) * SUB, :] = h.astype(jnp.bfloat16)


def _run_ffn_in(x_bf, w1, w3, be, bv):
    grid = (NB, NF1 + 1)

    def x_map(b, f, be_r, bv_r):
        return (jnp.where(bv_r[b] > 0, b, 0), 0)

    def w_map(b, f, be_r, bv_r):
        fc = jnp.minimum(f, NF1 - 1)
        return (be_r[b], 0, jnp.where(bv_r[b] > 0, fc, 0))

    def h_map(b, f, be_r, bv_r):
        return (b, jnp.maximum(f - 1, 0))

    return pl.pallas_call(
        _ffn_in_kernel,
        grid_spec=pltpu.PrefetchScalarGridSpec(
            num_scalar_prefetch=2,
            grid=grid,
            in_specs=[
                pl.BlockSpec((RB, D), x_map),
                pl.BlockSpec((1, D, FB), w_map),
                pl.BlockSpec((1, D, FB), w_map),
            ],
            out_specs=pl.BlockSpec((RB, FB), h_map),
            scratch_shapes=[
                pltpu.VMEM((2, D, FB), jnp.bfloat16),
                pltpu.VMEM((2, D, FB), jnp.bfloat16),
            ],
        ),
        out_shape=jax.ShapeDtypeStruct((NB * RB, F), jnp.bfloat16),
        compiler_params=pltpu.CompilerParams(
            dimension_semantics=("arbitrary", "arbitrary")),
    )(be, bv, x_bf, w1, w3)


def _ffn_out_kernel(be_ref, bv_ref, h_ref, w2_ref, y_ref, w2b_ref):
    b = pl.program_id(0)
    k = pl.program_id(1)
    valid = bv_ref[b]
    cur = jax.lax.rem(k, 2)
    prv = jax.lax.rem(k + 1, 2)

    @pl.when(k < NF2)
    def _():
        w2b_ref[cur] = w2_ref[0].astype(jnp.bfloat16)

    for s in range(RB // SUB):
        @pl.when((k > 0) & (s * SUB < valid))
        def _():
            hs = h_ref[s * SUB:(s + 1) * SUB, :]
            part = jax.lax.dot_general(
                hs, w2b_ref[prv], (((1,), (0,)), ((), ())),
                preferred_element_type=jnp.float32)
            prev = y_ref[s * SUB:(s + 1) * SUB, :]
            y_ref[s * SUB:(s + 1) * SUB, :] = jnp.where(k == 1, part,
                                                        prev + part)


def _run_ffn_out(h, w2, be, bv):
    grid = (NB, NF2 + 1)

    def h_map(b, k, be_r, bv_r):
        kc = jnp.maximum(k - 1, 0)
        return (b, jnp.where(bv_r[b] > 0, kc, 0))

    def w_map(b, k, be_r, bv_r):
        kc = jnp.minimum(k, NF2 - 1)
        return (be_r[b], jnp.where(bv_r[b] > 0, kc, 0), 0)

    def y_map(b, k, be_r, bv_r):
        return (b, 0)

    return pl.pallas_call(
        _ffn_out_kernel,
        grid_spec=pltpu.PrefetchScalarGridSpec(
            num_scalar_prefetch=2,
            grid=grid,
            in_specs=[
                pl.BlockSpec((RB, FK), h_map),
                pl.BlockSpec((1, FK, D), w_map),
            ],
            out_specs=pl.BlockSpec((RB, D), y_map),
            scratch_shapes=[
                pltpu.VMEM((2, FK, D), jnp.bfloat16),
            ],
        ),
        out_shape=jax.ShapeDtypeStruct((NB * RB, D), jnp.float32),
        compiler_params=pltpu.CompilerParams(
            dimension_semantics=("arbitrary", "arbitrary")),
    )(be, bv, h, w2)


def kernel(x, gate_w, w1, w3, w2):
    be, bv, dest, gws = _run_router(x, gate_w)
    d1 = dest[:, 0].reshape(1, T)
    d2 = dest[:, 1].reshape(1, T)
    g1 = gws[:, 0].reshape(1, T)
    g2 = gws[:, 1].reshape(1, T)
    dest_all = jnp.concatenate([d1, d2], axis=1)
    disp_idx = dest_all * QCS + jnp.arange(QCS, dtype=jnp.int32)[:, None]
    i1q = d1 * QCC + jnp.arange(QCC, dtype=jnp.int32)[:, None]
    i2q = d2 * QCC + jnp.arange(QCC, dtype=jnp.int32)[:, None]

    x_sorted = _run_dispatch(x, disp_idx)
    x_bf = x_sorted.reshape(NB * RB, D).astype(jnp.bfloat16)
    h = _run_ffn_in(x_bf, w1, w3, be, bv)
    y_sorted = _run_ffn_out(h, w2, be, bv)
    out = _run_combine(y_sorted.reshape(NB * RB * QCC, DCC), i1q, i2q, g1, g2)
    return out


# speedup vs baseline: 1.1290x; 1.0280x over previous
import functools

import jax
import jax.numpy as jnp
from jax.experimental import pallas as pl
from jax.experimental.pallas import tpu as pltpu
from jax.experimental.pallas import tpu_sc as plsc

T = 2048
D = 4096
F = 14336
E = 8
RB = 1024
SUB = 256
NB = (2 * T) // RB + (E - 1)
FB = 256
FK = 256
NEG = -1e30


TBR = 256


def _router1_kernel(x_ref, gw_ref, a1_ref, a2_ref, g_ref):
    logits = jax.lax.dot_general(
        x_ref[...], gw_ref[...], (((1,), (0,)), ((), ())),
        precision=jax.lax.Precision.DEFAULT,
        preferred_element_type=jnp.float32)
    iota_e = jax.lax.broadcasted_iota(jnp.int32, (TBR, E), 1)
    l1 = jnp.max(logits, axis=1, keepdims=True)
    i1 = jnp.min(jnp.where(logits == l1, iota_e, E), axis=1,
                 keepdims=True)
    masked = jnp.where(iota_e == i1, NEG, logits)
    l2 = jnp.max(masked, axis=1, keepdims=True)
    i2 = jnp.min(jnp.where(masked == l2, iota_e, E), axis=1, keepdims=True)
    g_ref[:, 0:1] = jax.nn.sigmoid(l1 - l2)
    g_ref[:, 1:2] = jax.nn.sigmoid(l2 - l1)
    a1_ref[...] = (iota_e == i1).astype(jnp.int32)
    a2_ref[...] = (iota_e == i2).astype(jnp.int32)


def _run_router1(x, gate_w):
    out_shapes = (
        jax.ShapeDtypeStruct((T, E), jnp.int32),
        jax.ShapeDtypeStruct((T, E), jnp.int32),
        jax.ShapeDtypeStruct((T, 2), jnp.float32),
    )
    return pl.pallas_call(
        _router1_kernel,
        grid=(T // TBR,),
        out_shape=out_shapes,
        in_specs=[
            pl.BlockSpec((TBR, D), lambda i: (i, 0)),
            pl.BlockSpec((D, E), lambda i: (0, 0)),
        ],
        out_specs=(
            pl.BlockSpec((TBR, E), lambda i: (i, 0)),
            pl.BlockSpec((TBR, E), lambda i: (i, 0)),
            pl.BlockSpec((TBR, 2), lambda i: (i, 0)),
        ),
        compiler_params=pltpu.CompilerParams(
            dimension_semantics=("arbitrary",)),
    )(x, gate_w)


def _router2_kernel(a1_ref, a2_ref, be_ref, bv_ref, dest_ref,
                    cnt_vmem, cnt_smem, sem):
    a1 = a1_ref[...]
    a2 = a2_ref[...]
    a = a1 + a2
    s = a
    sh = 1
    while sh < T:
        s = s + jnp.concatenate(
            [jnp.zeros((sh, E), jnp.int32), s[:T - sh, :]], axis=0)
        sh *= 2
    pos = s - a
    counts = s[T - 1:T, :]
    cnt_vmem[...] = counts
    copy = pltpu.make_async_copy(cnt_vmem, cnt_smem, sem)
    copy.start()
    copy.wait()

    def expert_step(e, blk):
        cnt = cnt_smem[0, e]
        nb = (cnt + RB - 1) // RB

        def blk_step(j, _):
            be_ref[blk + j] = e
            bv_ref[blk + j] = jnp.minimum(RB, cnt - j * RB)
            return 0
        jax.lax.fori_loop(0, nb, blk_step, 0, unroll=False)
        return blk + nb

    used = jax.lax.fori_loop(0, E, expert_step, 0, unroll=False)

    def pad_step(j, _):
        @pl.when(j >= used)
        def _():
            be_ref[j] = E - 1
            bv_ref[j] = 0
        return 0
    jax.lax.fori_loop(0, NB, pad_step, 0, unroll=False)

    nb_vec = (counts + (RB - 1)) // RB
    off = nb_vec
    sh = 1
    while sh < E:
        off = off + jnp.concatenate(
            [jnp.zeros((1, sh), jnp.int32), off[:, :E - sh]], axis=1)
        sh *= 2
    pado = (off - nb_vec) * RB

    dest_ref[:, 0:1] = jnp.sum(a1 * (pos + pado), axis=1, keepdims=True)
    dest_ref[:, 1:2] = jnp.sum(a2 * (pos + pado), axis=1, keepdims=True)


def _run_router2(a1, a2):
    out_shapes = (
        jax.ShapeDtypeStruct((NB,), jnp.int32),
        jax.ShapeDtypeStruct((NB,), jnp.int32),
        jax.ShapeDtypeStruct((T, 2), jnp.int32),
    )
    return pl.pallas_call(
        _router2_kernel,
        out_shape=out_shapes,
        in_specs=[
            pl.BlockSpec(memory_space=pltpu.VMEM),
            pl.BlockSpec(memory_space=pltpu.VMEM),
        ],
        out_specs=(
            pl.BlockSpec(memory_space=pltpu.SMEM),
            pl.BlockSpec(memory_space=pltpu.SMEM),
            pl.BlockSpec(memory_space=pltpu.VMEM),
        ),
        scratch_shapes=[
            pltpu.VMEM((1, E), jnp.int32),
            pltpu.SMEM((1, E), jnp.int32),
            pltpu.SemaphoreType.DMA,
        ],
    )(a1, a2)


def _run_router(x, gate_w):
    a1, a2, gws = _run_router1(x, gate_w)
    be, bv, dest = _run_router2(a1, a2)
    return be, bv, dest, gws


def _sc_mesh():
    return plsc.VectorSubcoreMesh(core_axis_name="core",
                                  subcore_axis_name="subcore",
                                  num_cores=2, num_subcores=16)


WIN = 128
DCS = 256
QCS = D // DCS
DCC = 256
QCC = D // DCC


def _run_dispatch(x_words, idxs):
    @functools.partial(
        pl.kernel,
        out_type=jax.ShapeDtypeStruct((NB * RB * QCS, DCS), jnp.float32),
        mesh=_sc_mesh())
    def scatter_kernel(x_hbm, i_hbm, o_hbm):
        def body(x_vmem, i_vmem):
            pltpu.sync_copy(x_vmem, o_hbm.at[i_vmem.at[0]])

        pltpu.emit_pipeline(
            body,
            grid=(QCS, (2 * T) // WIN),
            in_specs=[
                pl.BlockSpec((WIN, DCS),
                             index_map=lambda q, i: (i % (T // WIN), q)),
                pl.BlockSpec((1, WIN), index_map=lambda q, i: (q, i)),
            ],
            out_specs=[],
            core_axis_name=("core", "subcore"),
            dimension_semantics=(pltpu.PARALLEL, pltpu.PARALLEL),
        )(x_hbm, i_hbm)

    return scatter_kernel(x_words, idxs)


def _run_combine(y_view, i1q, i2q, g1, g2):
    @functools.partial(
        pl.kernel,
        out_type=jax.ShapeDtypeStruct((T, D), jnp.float32),
        mesh=_sc_mesh(),
        scratch_types=[pltpu.VMEM((WIN, DCC), jnp.float32)])
    def combine_kernel(y_hbm, d1_hbm, d2_hbm, g1_hbm, g2_hbm, o_hbm, tmp):
        def body(i1_vmem, i2_vmem, g1_vmem, g2_vmem, o_vmem):
            pltpu.sync_copy(y_hbm.at[i1_vmem.at[0]], o_vmem)
            pltpu.sync_copy(y_hbm.at[i2_vmem.at[0]], tmp)

            @pl.loop(0, WIN)
            def _(i):
                ga = g1_vmem[0, pl.ds(i, 1)][0]
                gb = g2_vmem[0, pl.ds(i, 1)][0]

                @pl.loop(0, DCC, step=16)
                def _(c):
                    o_vmem.at[i, pl.ds(c, 16)][...] = (
                        ga * o_vmem.at[i, pl.ds(c, 16)][...]
                        + gb * tmp.at[i, pl.ds(c, 16)][...])

        iq_spec = pl.BlockSpec((1, WIN), index_map=lambda q, i: (q, i))
        g_spec = pl.BlockSpec((1, WIN), index_map=lambda q, i: (0, i))
        pltpu.emit_pipeline(
            body,
            grid=(QCC, T // WIN),
            in_specs=[iq_spec, iq_spec, g_spec, g_spec],
            out_specs=[pl.BlockSpec((WIN, DCC),
                                    index_map=lambda q, i: (i, q))],
            core_axis_name=("core", "subcore"),
            dimension_semantics=(pltpu.PARALLEL, pltpu.PARALLEL),
        )(d1_hbm, d2_hbm, g1_hbm, g2_hbm, o_hbm)

    return combine_kernel(y_view, i1q, i2q, g1, g2)


def _cast_x_kernel(x_ref, o_ref):
    o_ref[...] = x_ref[...].astype(jnp.bfloat16)


def _run_cast_x(x_sorted):
    return pl.pallas_call(
        _cast_x_kernel,
        grid=(NB * 2,),
        in_specs=[pl.BlockSpec((RB // 2, D), lambda i: (i, 0))],
        out_specs=pl.BlockSpec((RB // 2, D), lambda i: (i, 0)),
        out_shape=jax.ShapeDtypeStruct((NB * RB, D), jnp.bfloat16),
        compiler_params=pltpu.CompilerParams(
            dimension_semantics=("arbitrary",)),
    )(x_sorted)


def _ffn_in_kernel(be_ref, bv_ref, x_ref, w1_ref, w3_ref, h_ref):
    b = pl.program_id(0)
    valid = bv_ref[b]
    w1b = w1_ref[0].astype(jnp.bfloat16)
    w3b = w3_ref[0].astype(jnp.bfloat16)
    for s in range(RB // SUB):
        @pl.when(s * SUB < valid)
        def _():
            xs = x_ref[s * SUB:(s + 1) * SUB, :].astype(jnp.bfloat16)
            a = jax.lax.dot_general(
                xs, w1b, (((1,), (0,)), ((), ())),
                preferred_element_type=jnp.float32)
            u = jax.lax.dot_general(
                xs, w3b, (((1,), (0,)), ((), ())),
                preferred_element_type=jnp.float32)
            h = (a * jax.nn.sigmoid(a)) * u
            h_ref[s * SUB:(s + 1) * SUB, :] = h.astype(jnp.bfloat16)


def _run_ffn_in(x_sorted, w1, w3, be, bv):
    grid = (NB, F // FB)

    def x_map(b, f, be_r, bv_r):
        return (jnp.where(bv_r[b] > 0, b, 0), 0)

    def w_map(b, f, be_r, bv_r):
        return (be_r[b], 0, jnp.where(bv_r[b] > 0, f, 0))

    def h_map(b, f, be_r, bv_r):
        return (b, f)

    return pl.pallas_call(
        _ffn_in_kernel,
        grid_spec=pltpu.PrefetchScalarGridSpec(
            num_scalar_prefetch=2,
            grid=grid,
            in_specs=[
                pl.BlockSpec((RB, D), x_map),
                pl.BlockSpec((1, D, FB), w_map),
                pl.BlockSpec((1, D, FB), w_map),
            ],
            out_specs=pl.BlockSpec((RB, FB), h_map),
        ),
        out_shape=jax.ShapeDtypeStruct((NB * RB, F), jnp.bfloat16),
        compiler_params=pltpu.CompilerParams(
            dimension_semantics=("arbitrary", "arbitrary")),
    )(be, bv, x_sorted, w1, w3)


def _ffn_out_kernel(be_ref, bv_ref, h_ref, w2_ref, y_ref):
    b = pl.program_id(0)
    k = pl.program_id(1)
    valid = bv_ref[b]
    w2b = w2_ref[0].astype(jnp.bfloat16)
    for s in range(RB // SUB):
        @pl.when(s * SUB < valid)
        def _():
            hs = h_ref[s * SUB:(s + 1) * SUB, :]
            part = jax.lax.dot_general(
                hs, w2b, (((1,), (0,)), ((), ())),
                preferred_element_type=jnp.float32)
            prev = y_ref[s * SUB:(s + 1) * SUB, :]
            y_ref[s * SUB:(s + 1) * SUB, :] = jnp.where(k == 0, part,
                                                        prev + part)


def _run_ffn_out(h, w2, be, bv):
    grid = (NB, F // FK)

    def h_map(b, k, be_r, bv_r):
        return (b, jnp.where(bv_r[b] > 0, k, 0))

    def w_map(b, k, be_r, bv_r):
        return (be_r[b], jnp.where(bv_r[b] > 0, k, 0), 0)

    def y_map(b, k, be_r, bv_r):
        return (b, 0)

    return pl.pallas_call(
        _ffn_out_kernel,
        grid_spec=pltpu.PrefetchScalarGridSpec(
            num_scalar_prefetch=2,
            grid=grid,
            in_specs=[
                pl.BlockSpec((RB, FK), h_map),
                pl.BlockSpec((1, FK, D), w_map),
            ],
            out_specs=pl.BlockSpec((RB, D), y_map),
        ),
        out_shape=jax.ShapeDtypeStruct((NB * RB, D), jnp.float32),
        compiler_params=pltpu.CompilerParams(
            dimension_semantics=("arbitrary", "arbitrary")),
    )(be, bv, h, w2)


def kernel(x, gate_w, w1, w3, w2):
    be, bv, dest, gws = _run_router(x, gate_w)
    d1 = dest[:, 0].reshape(1, T)
    d2 = dest[:, 1].reshape(1, T)
    g1 = gws[:, 0].reshape(1, T)
    g2 = gws[:, 1].reshape(1, T)
    dest_all = jnp.concatenate([d1, d2], axis=1)
    disp_idx = dest_all * QCS + jnp.arange(QCS, dtype=jnp.int32)[:, None]
    i1q = d1 * QCC + jnp.arange(QCC, dtype=jnp.int32)[:, None]
    i2q = d2 * QCC + jnp.arange(QCC, dtype=jnp.int32)[:, None]

    x_sorted = _run_dispatch(x, disp_idx).reshape(NB * RB, D)
    h = _run_ffn_in(x_sorted, w1, w3, be, bv)
    y_sorted = _run_ffn_out(h, w2, be, bv)
    out = _run_combine(y_sorted.reshape(NB * RB * QCC, DCC), i1q, i2q, g1, g2)
    return out


# speedup vs baseline: 1.2436x; 1.1015x over previous
import functools

import jax
import jax.numpy as jnp
from jax.experimental import pallas as pl
from jax.experimental.pallas import tpu as pltpu
from jax.experimental.pallas import tpu_sc as plsc

T = 2048
D = 4096
F = 14336
E = 8
RB = 1024
SUB = 256
NB = (2 * T) // RB + (E - 1)
FB = 256
FK = 256
NEG = -1e30


TBR = 256


def _router1_kernel(x_ref, gw_ref, a1_ref, a2_ref, g_ref):
    logits = jax.lax.dot_general(
        x_ref[...], gw_ref[...], (((1,), (0,)), ((), ())),
        precision=jax.lax.Precision.DEFAULT,
        preferred_element_type=jnp.float32)
    iota_e = jax.lax.broadcasted_iota(jnp.int32, (TBR, E), 1)
    l1 = jnp.max(logits, axis=1, keepdims=True)
    i1 = jnp.min(jnp.where(logits == l1, iota_e, E), axis=1,
                 keepdims=True)
    masked = jnp.where(iota_e == i1, NEG, logits)
    l2 = jnp.max(masked, axis=1, keepdims=True)
    i2 = jnp.min(jnp.where(masked == l2, iota_e, E), axis=1, keepdims=True)
    g_ref[:, 0:1] = jax.nn.sigmoid(l1 - l2)
    g_ref[:, 1:2] = jax.nn.sigmoid(l2 - l1)
    a1_ref[...] = (iota_e == i1).astype(jnp.int32)
    a2_ref[...] = (iota_e == i2).astype(jnp.int32)


def _run_router1(x, gate_w):
    out_shapes = (
        jax.ShapeDtypeStruct((T, E), jnp.int32),
        jax.ShapeDtypeStruct((T, E), jnp.int32),
        jax.ShapeDtypeStruct((T, 2), jnp.float32),
    )
    return pl.pallas_call(
        _router1_kernel,
        grid=(T // TBR,),
        out_shape=out_shapes,
        in_specs=[
            pl.BlockSpec((TBR, D), lambda i: (i, 0)),
            pl.BlockSpec((D, E), lambda i: (0, 0)),
        ],
        out_specs=(
            pl.BlockSpec((TBR, E), lambda i: (i, 0)),
            pl.BlockSpec((TBR, E), lambda i: (i, 0)),
            pl.BlockSpec((TBR, 2), lambda i: (i, 0)),
        ),
        compiler_params=pltpu.CompilerParams(
            dimension_semantics=("arbitrary",)),
    )(x, gate_w)


def _router2_kernel(a1_ref, a2_ref, be_ref, bv_ref, dest_ref,
                    cnt_vmem, cnt_smem, sem):
    a1 = a1_ref[...]
    a2 = a2_ref[...]
    a = a1 + a2
    s = a
    sh = 1
    while sh < T:
        s = s + jnp.concatenate(
            [jnp.zeros((sh, E), jnp.int32), s[:T - sh, :]], axis=0)
        sh *= 2
    pos = s - a
    counts = s[T - 1:T, :]
    cnt_vmem[...] = counts
    copy = pltpu.make_async_copy(cnt_vmem, cnt_smem, sem)
    copy.start()
    copy.wait()

    def expert_step(e, blk):
        cnt = cnt_smem[0, e]
        nb = (cnt + RB - 1) // RB

        def blk_step(j, _):
            be_ref[blk + j] = e
            bv_ref[blk + j] = jnp.minimum(RB, cnt - j * RB)
            return 0
        jax.lax.fori_loop(0, nb, blk_step, 0, unroll=False)
        return blk + nb

    used = jax.lax.fori_loop(0, E, expert_step, 0, unroll=False)

    def pad_step(j, _):
        @pl.when(j >= used)
        def _():
            be_ref[j] = E - 1
            bv_ref[j] = 0
        return 0
    jax.lax.fori_loop(0, NB, pad_step, 0, unroll=False)

    nb_vec = (counts + (RB - 1)) // RB
    off = nb_vec
    sh = 1
    while sh < E:
        off = off + jnp.concatenate(
            [jnp.zeros((1, sh), jnp.int32), off[:, :E - sh]], axis=1)
        sh *= 2
    pado = (off - nb_vec) * RB

    dest_ref[:, 0:1] = jnp.sum(a1 * (pos + pado), axis=1, keepdims=True)
    dest_ref[:, 1:2] = jnp.sum(a2 * (pos + pado), axis=1, keepdims=True)


def _run_router2(a1, a2):
    out_shapes = (
        jax.ShapeDtypeStruct((NB,), jnp.int32),
        jax.ShapeDtypeStruct((NB,), jnp.int32),
        jax.ShapeDtypeStruct((T, 2), jnp.int32),
    )
    return pl.pallas_call(
        _router2_kernel,
        out_shape=out_shapes,
        in_specs=[
            pl.BlockSpec(memory_space=pltpu.VMEM),
            pl.BlockSpec(memory_space=pltpu.VMEM),
        ],
        out_specs=(
            pl.BlockSpec(memory_space=pltpu.SMEM),
            pl.BlockSpec(memory_space=pltpu.SMEM),
            pl.BlockSpec(memory_space=pltpu.VMEM),
        ),
        scratch_shapes=[
            pltpu.VMEM((1, E), jnp.int32),
            pltpu.SMEM((1, E), jnp.int32),
            pltpu.SemaphoreType.DMA,
        ],
    )(a1, a2)


def _run_router(x, gate_w):
    a1, a2, gws = _run_router1(x, gate_w)
    be, bv, dest = _run_router2(a1, a2)
    return be, bv, dest, gws


def _sc_mesh():
    return plsc.VectorSubcoreMesh(core_axis_name="core",
                                  subcore_axis_name="subcore",
                                  num_cores=2, num_subcores=16)


WIN = 128
DCS = 256
QCS = D // DCS
DCC = 256
QCC = D // DCC


def _run_dispatch(x_words, idxs):
    @functools.partial(
        pl.kernel,
        out_type=jax.ShapeDtypeStruct((NB * RB * QCS, DCS), jnp.float32),
        mesh=_sc_mesh())
    def scatter_kernel(x_hbm, i_hbm, o_hbm):
        def body(x_vmem, i_vmem):
            pltpu.sync_copy(x_vmem, o_hbm.at[i_vmem.at[0]])

        pltpu.emit_pipeline(
            body,
            grid=(QCS, (2 * T) // WIN),
            in_specs=[
                pl.BlockSpec((WIN, DCS),
                             index_map=lambda q, i: (i % (T // WIN), q)),
                pl.BlockSpec((1, WIN), index_map=lambda q, i: (q, i)),
            ],
            out_specs=[],
            core_axis_name=("core", "subcore"),
            dimension_semantics=(pltpu.PARALLEL, pltpu.PARALLEL),
        )(x_hbm, i_hbm)

    return scatter_kernel(x_words, idxs)


CW = 8
NWORK = 32
TPW = T // NWORK
NCH = TPW // CW


def _run_combine(y_sorted, d1, d2, g1, g2):
    @functools.partial(
        pl.kernel,
        out_type=jax.ShapeDtypeStruct((T, D), jnp.float32),
        mesh=_sc_mesh(),
        scratch_types=[
            pltpu.VMEM((TPW,), jnp.int32),
            pltpu.VMEM((TPW,), jnp.int32),
            pltpu.VMEM((TPW,), jnp.float32),
            pltpu.VMEM((TPW,), jnp.float32),
            pltpu.VMEM((CW, D), jnp.float32),
            pltpu.VMEM((CW, D), jnp.float32),
            pltpu.SemaphoreType.DMA,
            pltpu.SemaphoreType.DMA,
        ])
    def combine_kernel(y_hbm, d1_hbm, d2_hbm, g1_hbm, g2_hbm, o_hbm,
                       i1_v, i2_v, g1_v, g2_v, r1_v, r2_v, s1, s2):
        wid = jax.lax.axis_index("subcore") * 2 + jax.lax.axis_index("core")
        base = wid * TPW
        pltpu.sync_copy(d1_hbm.at[0, pl.ds(base, TPW)], i1_v)
        pltpu.sync_copy(d2_hbm.at[0, pl.ds(base, TPW)], i2_v)
        pltpu.sync_copy(g1_hbm.at[0, pl.ds(base, TPW)], g1_v)
        pltpu.sync_copy(g2_hbm.at[0, pl.ds(base, TPW)], g2_v)

        @pl.loop(0, NCH)
        def _(j):
            c1 = pltpu.make_async_copy(
                y_hbm.at[i1_v.at[pl.ds(j * CW, CW)]], r1_v, s1)
            c2 = pltpu.make_async_copy(
                y_hbm.at[i2_v.at[pl.ds(j * CW, CW)]], r2_v, s2)
            c1.start()
            c2.start()
            c1.wait()
            c2.wait()

            @pl.loop(0, CW)
            def _(i):
                ga = g1_v[pl.ds(j * CW + i, 1)][0]
                gb = g2_v[pl.ds(j * CW + i, 1)][0]

                @pl.loop(0, D, step=16)
                def _(c):
                    r1_v.at[i, pl.ds(c, 16)][...] = (
                        ga * r1_v.at[i, pl.ds(c, 16)][...]
                        + gb * r2_v.at[i, pl.ds(c, 16)][...])

            pltpu.sync_copy(r1_v, o_hbm.at[pl.ds(base + j * CW, CW)])

    return combine_kernel(y_sorted, d1, d2, g1, g2)


def _cast_x_kernel(x_ref, o_ref):
    o_ref[...] = x_ref[...].astype(jnp.bfloat16)


def _run_cast_x(x_sorted):
    return pl.pallas_call(
        _cast_x_kernel,
        grid=(NB * 2,),
        in_specs=[pl.BlockSpec((RB // 2, D), lambda i: (i, 0))],
        out_specs=pl.BlockSpec((RB // 2, D), lambda i: (i, 0)),
        out_shape=jax.ShapeDtypeStruct((NB * RB, D), jnp.bfloat16),
        compiler_params=pltpu.CompilerParams(
            dimension_semantics=("arbitrary",)),
    )(x_sorted)


def _ffn_in_kernel(be_ref, bv_ref, x_ref, w1_ref, w3_ref, h_ref):
    b = pl.program_id(0)
    valid = bv_ref[b]
    w1b = w1_ref[0].astype(jnp.bfloat16)
    w3b = w3_ref[0].astype(jnp.bfloat16)
    for s in range(RB // SUB):
        @pl.when(s * SUB < valid)
        def _():
            xs = x_ref[s * SUB:(s + 1) * SUB, :].astype(jnp.bfloat16)
            a = jax.lax.dot_general(
                xs, w1b, (((1,), (0,)), ((), ())),
                preferred_element_type=jnp.float32)
            u = jax.lax.dot_general(
                xs, w3b, (((1,), (0,)), ((), ())),
                preferred_element_type=jnp.float32)
            h = (a * jax.nn.sigmoid(a)) * u
            h_ref[s * SUB:(s + 1) * SUB, :] = h.astype(jnp.bfloat16)


def _run_ffn_in(x_sorted, w1, w3, be, bv):
    grid = (NB, F // FB)

    def x_map(b, f, be_r, bv_r):
        return (jnp.where(bv_r[b] > 0, b, 0), 0)

    def w_map(b, f, be_r, bv_r):
        return (be_r[b], 0, jnp.where(bv_r[b] > 0, f, 0))

    def h_map(b, f, be_r, bv_r):
        return (b, f)

    return pl.pallas_call(
        _ffn_in_kernel,
        grid_spec=pltpu.PrefetchScalarGridSpec(
            num_scalar_prefetch=2,
            grid=grid,
            in_specs=[
                pl.BlockSpec((RB, D), x_map),
                pl.BlockSpec((1, D, FB), w_map),
                pl.BlockSpec((1, D, FB), w_map),
            ],
            out_specs=pl.BlockSpec((RB, FB), h_map),
        ),
        out_shape=jax.ShapeDtypeStruct((NB * RB, F), jnp.bfloat16),
        compiler_params=pltpu.CompilerParams(
            dimension_semantics=("arbitrary", "arbitrary")),
    )(be, bv, x_sorted, w1, w3)


def _ffn_out_kernel(be_ref, bv_ref, h_ref, w2_ref, y_ref):
    b = pl.program_id(0)
    k = pl.program_id(1)
    valid = bv_ref[b]
    w2b = w2_ref[0].astype(jnp.bfloat16)
    for s in range(RB // SUB):
        @pl.when(s * SUB < valid)
        def _():
            hs = h_ref[s * SUB:(s + 1) * SUB, :]
            part = jax.lax.dot_general(
                hs, w2b, (((1,), (0,)), ((), ())),
                preferred_element_type=jnp.float32)
            prev = y_ref[s * SUB:(s + 1) * SUB, :]
            y_ref[s * SUB:(s + 1) * SUB, :] = jnp.where(k == 0, part,
                                                        prev + part)


def _run_ffn_out(h, w2, be, bv):
    grid = (NB, F // FK)

    def h_map(b, k, be_r, bv_r):
        return (b, jnp.where(bv_r[b] > 0, k, 0))

    def w_map(b, k, be_r, bv_r):
        return (be_r[b], jnp.where(bv_r[b] > 0, k, 0), 0)

    def y_map(b, k, be_r, bv_r):
        return (b, 0)

    return pl.pallas_call(
        _ffn_out_kernel,
        grid_spec=pltpu.PrefetchScalarGridSpec(
            num_scalar_prefetch=2,
            grid=grid,
            in_specs=[
                pl.BlockSpec((RB, FK), h_map),
                pl.BlockSpec((1, FK, D), w_map),
            ],
            out_specs=pl.BlockSpec((RB, D), y_map),
        ),
        out_shape=jax.ShapeDtypeStruct((NB * RB, D), jnp.float32),
        compiler_params=pltpu.CompilerParams(
            dimension_semantics=("arbitrary", "arbitrary")),
    )(be, bv, h, w2)


def kernel(x, gate_w, w1, w3, w2):
    be, bv, dest, gws = _run_router(x, gate_w)
    d1 = dest[:, 0].reshape(1, T)
    d2 = dest[:, 1].reshape(1, T)
    g1 = gws[:, 0].reshape(1, T)
    g2 = gws[:, 1].reshape(1, T)
    dest_all = jnp.concatenate([d1, d2], axis=1)
    disp_idx = dest_all * QCS + jnp.arange(QCS, dtype=jnp.int32)[:, None]

    x_sorted = _run_dispatch(x, disp_idx).reshape(NB * RB, D)
    h = _run_ffn_in(x_sorted, w1, w3, be, bv)
    y_sorted = _run_ffn_out(h, w2, be, bv)
    out = _run_combine(y_sorted, d1, d2, g1, g2)
    return out
